# baseline (speedup 1.0000x reference)
import functools

import jax
import jax.numpy as jnp
from jax.experimental import pallas as pl
from jax.experimental.pallas import tpu as pltpu

_B, _N, _D = 8, 512, 256
_K = 8


def _topk_keep(P):
    iota = jax.lax.broadcasted_iota(jnp.int32, P.shape, 1).astype(jnp.float32)
    big = jnp.float32(2.0 * _N)
    work = P
    idxs = []
    for t in range(_K):
        m = jnp.max(work, axis=1, keepdims=True)
        idx = jnp.min(jnp.where(work == m, iota, big), axis=1, keepdims=True)
        idxs.append(idx)
        if t + 1 < _K:
            work = jnp.where(iota == idx, -jnp.inf, work)
    keep = iota == idxs[0]
    for idx in idxs[1:]:
        keep = keep | (iota == idx)
    return jnp.where(keep, P, 0.0)


def _dotT(x, y):
    return jax.lax.dot_general(
        x, y, dimension_numbers=(((1,), (1,)), ((), ())),
        preferred_element_type=jnp.float32)


def _body(a_ref, b_ref, t1_ref, t2_ref, ta_ref, tb_ref,
          oa_ref, ob_ref, oi_ref):
    sig = jax.nn.sigmoid
    a = a_ref[0]
    b = b_ref[0]
    t1 = t1_ref[...]
    t2 = t2_ref[...]
    ta = ta_ref[...]
    tb = tb_ref[...]

    pa = sig(_dotT(a * t1, a * t2))
    oa_ref[0] = _topk_keep(jax.nn.relu(pa - pa.T))

    pb = sig(_dotT(b * t1, b * t2))
    ob_ref[0] = _topk_keep(jax.nn.relu(pb - pb.T))

    oi_ref[0] = _topk_keep(sig(_dotT(a * ta, b * tb)))


@functools.partial(jax.jit, static_argnames=())
def kernel(tensor_a, tensor_b, theta1_intra, theta2_intra,
           theta_a_inter, theta_b_inter):
    t1 = theta1_intra.reshape(1, _D)
    t2 = theta2_intra.reshape(1, _D)
    ta = theta_a_inter.reshape(1, _D)
    tb = theta_b_inter.reshape(1, _D)

    batch_spec = pl.BlockSpec((1, _N, _D), lambda i: (i, 0, 0))
    theta_spec = pl.BlockSpec((1, _D), lambda i: (0, 0))
    out_spec = pl.BlockSpec((1, _N, _N), lambda i: (i, 0, 0))
    out_shape = jax.ShapeDtypeStruct((_B, _N, _N), jnp.float32)

    return pl.pallas_call(
        _body,
        grid=(_B,),
        in_specs=[batch_spec, batch_spec,
                  theta_spec, theta_spec, theta_spec, theta_spec],
        out_specs=[out_spec, out_spec, out_spec],
        out_shape=[out_shape, out_shape, out_shape],
        compiler_params=pltpu.CompilerParams(
            dimension_semantics=("parallel",)),
    )(tensor_a, tensor_b, t1, t2, ta, tb)


# speedup vs baseline: 1.0668x; 1.0668x over previous
import functools

import jax
import jax.numpy as jnp
from jax.experimental import pallas as pl
from jax.experimental.pallas import tpu as pltpu

_B, _N, _D = 8, 512, 256
_K = 8


def _topk_keep(P, U):
    work = P
    kf = jnp.float32(_K)
    cum = jnp.zeros((P.shape[0], 1), jnp.float32)
    thr = jnp.full((P.shape[0], 1), -jnp.inf, jnp.float32)
    need = jnp.full((P.shape[0], 1), kf, jnp.float32)
    for t in range(_K):
        m = jnp.max(work, axis=1, keepdims=True)
        eqm = work == m
        c = jnp.sum(eqm.astype(jnp.float32), axis=1, keepdims=True)
        open_ = cum < kf
        thr = jnp.where(open_, m, thr)
        need = jnp.where(open_, kf - cum, need)
        cum = cum + c
        if t + 1 < _K:
            work = jnp.where(eqm, -jnp.inf, work)
    gt = P > thr
    eqt = P == thr
    pref = jax.lax.dot_general(
        eqt.astype(jnp.float32), U,
        dimension_numbers=(((1,), (0,)), ((), ())),
        preferred_element_type=jnp.float32)
    keep = gt | (eqt & (pref <= need))
    return jnp.where(keep, P, 0.0)


def _dotT(x, y):
    return jax.lax.dot_general(
        x, y, dimension_numbers=(((1,), (1,)), ((), ())),
        preferred_element_type=jnp.float32)


def _body(a_ref, b_ref, t1_ref, t2_ref, ta_ref, tb_ref,
          oa_ref, ob_ref, oi_ref):
    sig = jax.nn.sigmoid
    a = a_ref[0]
    b = b_ref[0]
    U = (jax.lax.broadcasted_iota(jnp.int32, (_N, _N), 0)
         <= jax.lax.broadcasted_iota(jnp.int32, (_N, _N), 1)).astype(jnp.float32)
    t1 = t1_ref[...]
    t2 = t2_ref[...]
    ta = ta_ref[...]
    tb = tb_ref[...]

    pa = sig(_dotT(a * t1, a * t2))
    oa_ref[0] = _topk_keep(jax.nn.relu(pa - pa.T), U)

    pb = sig(_dotT(b * t1, b * t2))
    ob_ref[0] = _topk_keep(jax.nn.relu(pb - pb.T), U)

    oi_ref[0] = _topk_keep(sig(_dotT(a * ta, b * tb)), U)


@functools.partial(jax.jit, static_argnames=())
def kernel(tensor_a, tensor_b, theta1_intra, theta2_intra,
           theta_a_inter, theta_b_inter):
    t1 = theta1_intra.reshape(1, _D)
    t2 = theta2_intra.reshape(1, _D)
    ta = theta_a_inter.reshape(1, _D)
    tb = theta_b_inter.reshape(1, _D)

    batch_spec = pl.BlockSpec((1, _N, _D), lambda i: (i, 0, 0))
    theta_spec = pl.BlockSpec((1, _D), lambda i: (0, 0))
    out_spec = pl.BlockSpec((1, _N, _N), lambda i: (i, 0, 0))
    out_shape = jax.ShapeDtypeStruct((_B, _N, _N), jnp.float32)

    return pl.pallas_call(
        _body,
        grid=(_B,),
        in_specs=[batch_spec, batch_spec,
                  theta_spec, theta_spec, theta_spec, theta_spec],
        out_specs=[out_spec, out_spec, out_spec],
        out_shape=[out_shape, out_shape, out_shape],
        compiler_params=pltpu.CompilerParams(
            dimension_semantics=("parallel",)),
    )(tensor_a, tensor_b, t1, t2, ta, tb)
